# baseline (speedup 1.0000x reference)
import functools

import jax
import jax.numpy as jnp
from jax import lax
from jax.experimental import pallas as pl
from jax.experimental.pallas import tpu as pltpu
from jax.experimental.pallas import tpu_sc as plsc

B = 16384
D = 64
L = 16
NC = 2
NS = 16
NW = NC * NS
BPW = B // NW
CHUNK = 128
NCHUNK = BPW // CHUNK
LAG = 8

NU = 1000000
NI = 100000
W = 512
NB_MAIN = NU // W
TAILW = NU - NB_MAIN * W
NBLK = NB_MAIN + 1
BLK_PER_W = NB_MAIN // NW
LEN_P = NBLK * D * W


def _rp_body(pt_hbm, out_hbm, buf, semi, semo):
    wid = lax.axis_index("s") * NC + lax.axis_index("c")
    c0 = wid * BLK_PER_W

    ins = [None] * BLK_PER_W
    outs = [None] * BLK_PER_W
    for t in range(BLK_PER_W):
        if t >= 2:
            outs[t - 2].wait()
        off = pl.multiple_of((c0 + t) * W, W)
        ins[t] = pltpu.async_copy(pt_hbm.at[:, pl.ds(off, W)],
                                  buf.at[t % 2], semi)
        if t >= 1:
            ins[t - 1].wait()
            outs[t - 1] = pltpu.async_copy(
                buf.at[(t - 1) % 2],
                out_hbm.at[:, pl.ds((c0 + t - 1) * W, W)], semo)
    ins[BLK_PER_W - 1].wait()
    outs[BLK_PER_W - 1] = pltpu.async_copy(
        buf.at[(BLK_PER_W - 1) % 2],
        out_hbm.at[:, pl.ds((c0 + BLK_PER_W - 1) * W, W)], semo)
    outs[BLK_PER_W - 2].wait()
    outs[BLK_PER_W - 1].wait()

    @pl.when(wid == 0)
    def _():
        pltpu.sync_copy(pt_hbm.at[:, pl.ds(NB_MAIN * W - W, W)], buf.at[0])
        pltpu.sync_copy(buf.at[0],
                        out_hbm.at[:, pl.ds(NB_MAIN * W - W, W)])


_rp = functools.partial(
    pl.kernel,
    out_type=jax.ShapeDtypeStruct((D, NU), jnp.float32),
    mesh=plsc.VectorSubcoreMesh(core_axis_name="c", subcore_axis_name="s"),
    compiler_params=pltpu.CompilerParams(
        needs_layout_passes=False, use_tc_tiling_on_sc=True),
    scratch_types=[
        pltpu.VMEM((2, D, W), jnp.float32),
        pltpu.SemaphoreType.DMA,
        pltpu.SemaphoreType.DMA,
    ],
)(_rp_body)


def _mf_body(u_hbm, i_hbm, pt_hbm, qt_hbm, out_hbm,
             uidx_v, iidx_v, pv, qv, out_v, semp, semq):
    wid = lax.axis_index("s") * NC + lax.axis_index("c")
    base = wid * BPW

    pltpu.sync_copy(u_hbm.at[wid], uidx_v)
    pltpu.sync_copy(i_hbm.at[wid], iidx_v)

    handles = [None] * D
    for k in range(D):
        ph = []
        for j in range(NCHUNK):
            sl = pl.ds(j * CHUNK, CHUNK)
            ph.append(pltpu.async_copy(
                pt_hbm.at[k].at[uidx_v.at[j]], pv.at[k, sl], semp))
            ph.append(pltpu.async_copy(
                qt_hbm.at[k].at[iidx_v.at[j]], qv.at[k, sl], semq))
        handles[k] = ph
        if k >= LAG:
            for h in handles[k - LAG]:
                h.wait()
    for k in range(D - LAG, D):
        for h in handles[k]:
            h.wait()

    def group(g, _):
        sl = pl.ds(g * L, L)
        acc = pv[0, sl] * qv[0, sl]
        for k in range(1, D):
            acc = acc + pv[k, sl] * qv[k, sl]
        out_v[sl] = acc
        return _

    lax.fori_loop(0, BPW // L, group, None)

    pltpu.sync_copy(out_v, out_hbm.at[pl.ds(base, BPW)])


_mf = functools.partial(
    pl.kernel,
    out_type=jax.ShapeDtypeStruct((B,), jnp.float32),
    mesh=plsc.VectorSubcoreMesh(core_axis_name="c", subcore_axis_name="s"),
    compiler_params=pltpu.CompilerParams(
        needs_layout_passes=False, use_tc_tiling_on_sc=False),
    scratch_types=[
        pltpu.VMEM((NCHUNK, CHUNK), jnp.int32),
        pltpu.VMEM((NCHUNK, CHUNK), jnp.int32),
        pltpu.VMEM((D, BPW), jnp.float32),
        pltpu.VMEM((D, BPW), jnp.float32),
        pltpu.VMEM((BPW,), jnp.float32),
        pltpu.SemaphoreType.DMA,
        pltpu.SemaphoreType.DMA,
    ],
)(_mf_body)


def _bias_body(u_hbm, i_hbm, bu_hbm, bi_hbm, out_hbm,
               uidx_v, iidx_v, buv_v, biv_v, sem):
    wid = lax.axis_index("s") * NC + lax.axis_index("c")
    base = wid * BPW

    pltpu.sync_copy(u_hbm.at[wid], uidx_v)
    pltpu.sync_copy(i_hbm.at[wid], iidx_v)

    copies = []
    for j in range(NCHUNK):
        sl = pl.ds(j * CHUNK, CHUNK)
        copies.append(pltpu.async_copy(bu_hbm.at[uidx_v.at[j]], buv_v.at[sl], sem))
        copies.append(pltpu.async_copy(bi_hbm.at[iidx_v.at[j]], biv_v.at[sl], sem))
    for c in copies:
        c.wait()

    for g in range(BPW // L):
        sl = pl.ds(g * L, L)
        buv_v[sl] = buv_v[sl] + biv_v[sl]

    pltpu.sync_copy(buv_v, out_hbm.at[pl.ds(base, BPW)])


_bias = functools.partial(
    pl.kernel,
    out_type=jax.ShapeDtypeStruct((B,), jnp.float32),
    mesh=plsc.VectorSubcoreMesh(core_axis_name="c", subcore_axis_name="s"),
    compiler_params=pltpu.CompilerParams(
        needs_layout_passes=False, use_tc_tiling_on_sc=False),
    scratch_types=[
        pltpu.VMEM((NCHUNK, CHUNK), jnp.int32),
        pltpu.VMEM((NCHUNK, CHUNK), jnp.int32),
        pltpu.VMEM((BPW,), jnp.float32),
        pltpu.VMEM((BPW,), jnp.float32),
        pltpu.SemaphoreType.DMA,
    ],
)(_bias_body)


@jax.jit
def kernel(u_idx, i_idx, mu, b_u, b_i, P, Q):
    u3 = u_idx.astype(jnp.int32).reshape(NW, NCHUNK, CHUNK)
    i3 = i_idx.astype(jnp.int32).reshape(NW, NCHUNK, CHUNK)
    pt_lin = _rp(P.T)
    pt_lin = lax.dynamic_update_slice(
        pt_lin, P[NB_MAIN * W:, :].T, (0, NB_MAIN * W))
    dot = _mf(u3, i3, pt_lin, Q.T)
    bias = _bias(u3, i3, b_u, b_i)
    return dot + bias + mu


# speedup vs baseline: 10.6295x; 10.6295x over previous
import functools

import jax
import jax.numpy as jnp
from jax import lax
from jax.experimental import pallas as pl
from jax.experimental.pallas import tpu as pltpu
from jax.experimental.pallas import tpu_sc as plsc

B = 16384
D = 64
L = 16
NC = 2
NS = 16
NW = NC * NS
BPW = B // NW
CHUNK = 128
NCHUNK = BPW // CHUNK
LAG = 8

NU = 1000000
NI = 100000
W = 512
NB_MAIN = NU // W
TAILW = NU - NB_MAIN * W
NBLK = NB_MAIN + 1
BLK_PER_W = NB_MAIN // NW
LEN_P = NBLK * D * W


def _rp_body(pt_hbm, out_hbm, buf, semi, semo):
    wid = lax.axis_index("s") * NC + lax.axis_index("c")
    c0 = wid * BLK_PER_W

    ins = [None] * BLK_PER_W
    outs = [None] * BLK_PER_W
    for t in range(BLK_PER_W):
        if t >= 2:
            outs[t - 2].wait()
        off = pl.multiple_of((c0 + t) * W, W)
        ins[t] = pltpu.async_copy(pt_hbm.at[:, pl.ds(off, W)],
                                  buf.at[t % 2], semi)
        if t >= 1:
            ins[t - 1].wait()
            outs[t - 1] = pltpu.async_copy(buf.at[(t - 1) % 2],
                                           out_hbm.at[c0 + t - 1], semo)
    ins[BLK_PER_W - 1].wait()
    outs[BLK_PER_W - 1] = pltpu.async_copy(
        buf.at[(BLK_PER_W - 1) % 2], out_hbm.at[c0 + BLK_PER_W - 1], semo)
    outs[BLK_PER_W - 2].wait()
    outs[BLK_PER_W - 1].wait()

    @pl.when(wid == 0)
    def _():
        pltpu.sync_copy(pt_hbm.at[:, pl.ds(NB_MAIN * W - W, W)], buf.at[0])
        pltpu.sync_copy(buf.at[0], out_hbm.at[NB_MAIN - 1])


_rp = functools.partial(
    pl.kernel,
    out_type=jax.ShapeDtypeStruct((NBLK, D, W), jnp.float32),
    mesh=plsc.VectorSubcoreMesh(core_axis_name="c", subcore_axis_name="s"),
    compiler_params=pltpu.CompilerParams(
        needs_layout_passes=False, use_tc_tiling_on_sc=True),
    scratch_types=[
        pltpu.VMEM((2, D, W), jnp.float32),
        pltpu.SemaphoreType.DMA,
        pltpu.SemaphoreType.DMA,
    ],
)(_rp_body)


def _mf_body(u_hbm, i_hbm, pl_hbm, qt_hbm, out_hbm,
             uidx_v, iidx_v, ubase_v, pv, qv, out_v, semp, semq):
    wid = lax.axis_index("s") * NC + lax.axis_index("c")
    base = wid * BPW

    pltpu.sync_copy(u_hbm.at[wid], uidx_v)
    pltpu.sync_copy(i_hbm.at[wid], iidx_v)

    sh9 = jnp.full((L,), 9, jnp.int32)
    sh15 = jnp.full((L,), 15, jnp.int32)
    m511 = jnp.full((L,), W - 1, jnp.int32)
    for c in range(NCHUNK):
        for g in range(CHUNK // L):
            sl = pl.ds(g * L, L)
            u = uidx_v[c, sl]
            ubase_v[c, sl] = lax.shift_left(
                lax.shift_right_logical(u, sh9), sh15
            ) + jnp.bitwise_and(u, m511)

    handles = [None] * D
    for k in range(D):
        ph = []
        for j in range(NCHUNK):
            sl = pl.ds(j * CHUNK, CHUNK)
            ph.append(pltpu.async_copy(
                pl_hbm.at[pl.ds(k * W, LEN_P - k * W)].at[ubase_v.at[j]],
                pv.at[k, sl], semp))
            ph.append(pltpu.async_copy(
                qt_hbm.at[k].at[iidx_v.at[j]], qv.at[k, sl], semq))
        handles[k] = ph
        if k >= LAG:
            for h in handles[k - LAG]:
                h.wait()
    for k in range(D - LAG, D):
        for h in handles[k]:
            h.wait()

    def group(g, _):
        sl = pl.ds(g * L, L)
        acc = pv[0, sl] * qv[0, sl]
        for k in range(1, D):
            acc = acc + pv[k, sl] * qv[k, sl]
        out_v[sl] = acc
        return _

    lax.fori_loop(0, BPW // L, group, None)

    pltpu.sync_copy(out_v, out_hbm.at[pl.ds(base, BPW)])


_mf = functools.partial(
    pl.kernel,
    out_type=jax.ShapeDtypeStruct((B,), jnp.float32),
    mesh=plsc.VectorSubcoreMesh(core_axis_name="c", subcore_axis_name="s"),
    compiler_params=pltpu.CompilerParams(
        needs_layout_passes=False, use_tc_tiling_on_sc=False),
    scratch_types=[
        pltpu.VMEM((NCHUNK, CHUNK), jnp.int32),
        pltpu.VMEM((NCHUNK, CHUNK), jnp.int32),
        pltpu.VMEM((NCHUNK, CHUNK), jnp.int32),
        pltpu.VMEM((D, BPW), jnp.float32),
        pltpu.VMEM((D, BPW), jnp.float32),
        pltpu.VMEM((BPW,), jnp.float32),
        pltpu.SemaphoreType.DMA,
        pltpu.SemaphoreType.DMA,
    ],
)(_mf_body)


def _bias_body(u_hbm, i_hbm, bu_hbm, bi_hbm, out_hbm,
               uidx_v, iidx_v, buv_v, biv_v, sem):
    wid = lax.axis_index("s") * NC + lax.axis_index("c")
    base = wid * BPW

    pltpu.sync_copy(u_hbm.at[wid], uidx_v)
    pltpu.sync_copy(i_hbm.at[wid], iidx_v)

    copies = []
    for j in range(NCHUNK):
        sl = pl.ds(j * CHUNK, CHUNK)
        copies.append(pltpu.async_copy(bu_hbm.at[uidx_v.at[j]], buv_v.at[sl], sem))
        copies.append(pltpu.async_copy(bi_hbm.at[iidx_v.at[j]], biv_v.at[sl], sem))
    for c in copies:
        c.wait()

    for g in range(BPW // L):
        sl = pl.ds(g * L, L)
        buv_v[sl] = buv_v[sl] + biv_v[sl]

    pltpu.sync_copy(buv_v, out_hbm.at[pl.ds(base, BPW)])


_bias = functools.partial(
    pl.kernel,
    out_type=jax.ShapeDtypeStruct((B,), jnp.float32),
    mesh=plsc.VectorSubcoreMesh(core_axis_name="c", subcore_axis_name="s"),
    compiler_params=pltpu.CompilerParams(
        needs_layout_passes=False, use_tc_tiling_on_sc=False),
    scratch_types=[
        pltpu.VMEM((NCHUNK, CHUNK), jnp.int32),
        pltpu.VMEM((NCHUNK, CHUNK), jnp.int32),
        pltpu.VMEM((BPW,), jnp.float32),
        pltpu.VMEM((BPW,), jnp.float32),
        pltpu.SemaphoreType.DMA,
    ],
)(_bias_body)


@jax.jit
def kernel(u_idx, i_idx, mu, b_u, b_i, P, Q):
    u3 = u_idx.astype(jnp.int32).reshape(NW, NCHUNK, CHUNK)
    i3 = i_idx.astype(jnp.int32).reshape(NW, NCHUNK, CHUNK)
    pl3 = _rp(P.T)
    tail = jnp.pad(P[NB_MAIN * W:, :].T, ((0, 0), (0, W - TAILW)))
    pl3 = pl3.at[NB_MAIN].set(tail)
    dot = _mf(u3, i3, pl3.reshape(LEN_P), Q.T)
    bias = _bias(u3, i3, b_u, b_i)
    return dot + bias + mu


# speedup vs baseline: 15.3614x; 1.4452x over previous
import functools

import jax
import jax.numpy as jnp
from jax import lax
from jax.experimental import pallas as pl
from jax.experimental.pallas import tpu as pltpu
from jax.experimental.pallas import tpu_sc as plsc

B = 16384
D = 64
L = 16
NC = 2
NS = 16
NW = NC * NS
BPW = B // NW
CHUNK = 128
NCHUNK = BPW // CHUNK
LAG = 8

NU = 1000000
NI = 100000
W = 512
NB_MAIN = NU // W
TAILW = NU - NB_MAIN * W
NBLK = NB_MAIN + 1
BLK_PER_W = NB_MAIN // NW
LEN_P = NBLK * D * W


def _rp_body(pt_hbm, out_hbm, buf, semi, semo):
    wid = lax.axis_index("s") * NC + lax.axis_index("c")
    c0 = wid * BLK_PER_W

    def drain_outs():
        for _ in range(D):
            pltpu.make_async_copy(buf.at[0, 0],
                                  out_hbm.at[pl.ds(0, W)], semo).wait()

    def block(t, _):
        c = c0 + t
        tb = jnp.bitwise_and(t, 1)
        in_h = pltpu.async_copy(
            pt_hbm.at[:, pl.ds(pl.multiple_of(c * W, W), W)],
            buf.at[tb], semi)
        @pl.when(t > 0)
        def _():
            drain_outs()
        in_h.wait()
        base = c * (D * W)
        for kk in range(D):
            pltpu.async_copy(buf.at[tb, kk],
                             out_hbm.at[pl.ds(base + kk * W, W)], semo)
        return _

    lax.fori_loop(0, BLK_PER_W, block, None)
    drain_outs()

    @pl.when(wid == 0)
    def _():
        c = NB_MAIN - 1
        pltpu.sync_copy(pt_hbm.at[:, pl.ds(c * W, W)], buf.at[0])
        for kk in range(D):
            pltpu.async_copy(buf.at[0, kk],
                             out_hbm.at[pl.ds(c * D * W + kk * W, W)], semo)
        drain_outs()


_rp = functools.partial(
    pl.kernel,
    out_type=jax.ShapeDtypeStruct((LEN_P,), jnp.float32),
    mesh=plsc.VectorSubcoreMesh(core_axis_name="c", subcore_axis_name="s"),
    compiler_params=pltpu.CompilerParams(
        needs_layout_passes=False, use_tc_tiling_on_sc=True),
    scratch_types=[
        pltpu.VMEM((2, D, W), jnp.float32),
        pltpu.SemaphoreType.DMA,
        pltpu.SemaphoreType.DMA,
    ],
)(_rp_body)


def _mf_body(u_hbm, i_hbm, pl_hbm, qt_hbm, out_hbm,
             uidx_v, iidx_v, ubase_v, pv, qv, out_v, semp, semq):
    wid = lax.axis_index("s") * NC + lax.axis_index("c")
    base = wid * BPW

    pltpu.sync_copy(u_hbm.at[wid], uidx_v)
    pltpu.sync_copy(i_hbm.at[wid], iidx_v)

    sh9 = jnp.full((L,), 9, jnp.int32)
    sh15 = jnp.full((L,), 15, jnp.int32)
    m511 = jnp.full((L,), W - 1, jnp.int32)
    for c in range(NCHUNK):
        for g in range(CHUNK // L):
            sl = pl.ds(g * L, L)
            u = uidx_v[c, sl]
            ubase_v[c, sl] = lax.shift_left(
                lax.shift_right_logical(u, sh9), sh15
            ) + jnp.bitwise_and(u, m511)

    handles = [None] * D
    for k in range(D):
        ph = []
        for j in range(NCHUNK):
            sl = pl.ds(j * CHUNK, CHUNK)
            ph.append(pltpu.async_copy(
                pl_hbm.at[pl.ds(k * W, LEN_P - k * W)].at[ubase_v.at[j]],
                pv.at[k, sl], semp))
            ph.append(pltpu.async_copy(
                qt_hbm.at[k].at[iidx_v.at[j]], qv.at[k, sl], semq))
        handles[k] = ph
        if k >= LAG:
            for h in handles[k - LAG]:
                h.wait()
    for k in range(D - LAG, D):
        for h in handles[k]:
            h.wait()

    def group(g, _):
        sl = pl.ds(g * L, L)
        acc = pv[0, sl] * qv[0, sl]
        for k in range(1, D):
            acc = acc + pv[k, sl] * qv[k, sl]
        out_v[sl] = acc
        return _

    lax.fori_loop(0, BPW // L, group, None)

    pltpu.sync_copy(out_v, out_hbm.at[pl.ds(base, BPW)])


_mf = functools.partial(
    pl.kernel,
    out_type=jax.ShapeDtypeStruct((B,), jnp.float32),
    mesh=plsc.VectorSubcoreMesh(core_axis_name="c", subcore_axis_name="s"),
    compiler_params=pltpu.CompilerParams(
        needs_layout_passes=False, use_tc_tiling_on_sc=False),
    scratch_types=[
        pltpu.VMEM((NCHUNK, CHUNK), jnp.int32),
        pltpu.VMEM((NCHUNK, CHUNK), jnp.int32),
        pltpu.VMEM((NCHUNK, CHUNK), jnp.int32),
        pltpu.VMEM((D, BPW), jnp.float32),
        pltpu.VMEM((D, BPW), jnp.float32),
        pltpu.VMEM((BPW,), jnp.float32),
        pltpu.SemaphoreType.DMA,
        pltpu.SemaphoreType.DMA,
    ],
)(_mf_body)


def _bias_body(u_hbm, i_hbm, bu_hbm, bi_hbm, out_hbm,
               uidx_v, iidx_v, buv_v, biv_v, sem):
    wid = lax.axis_index("s") * NC + lax.axis_index("c")
    base = wid * BPW

    pltpu.sync_copy(u_hbm.at[wid], uidx_v)
    pltpu.sync_copy(i_hbm.at[wid], iidx_v)

    copies = []
    for j in range(NCHUNK):
        sl = pl.ds(j * CHUNK, CHUNK)
        copies.append(pltpu.async_copy(bu_hbm.at[uidx_v.at[j]], buv_v.at[sl], sem))
        copies.append(pltpu.async_copy(bi_hbm.at[iidx_v.at[j]], biv_v.at[sl], sem))
    for c in copies:
        c.wait()

    for g in range(BPW // L):
        sl = pl.ds(g * L, L)
        buv_v[sl] = buv_v[sl] + biv_v[sl]

    pltpu.sync_copy(buv_v, out_hbm.at[pl.ds(base, BPW)])


_bias = functools.partial(
    pl.kernel,
    out_type=jax.ShapeDtypeStruct((B,), jnp.float32),
    mesh=plsc.VectorSubcoreMesh(core_axis_name="c", subcore_axis_name="s"),
    compiler_params=pltpu.CompilerParams(
        needs_layout_passes=False, use_tc_tiling_on_sc=False),
    scratch_types=[
        pltpu.VMEM((NCHUNK, CHUNK), jnp.int32),
        pltpu.VMEM((NCHUNK, CHUNK), jnp.int32),
        pltpu.VMEM((BPW,), jnp.float32),
        pltpu.VMEM((BPW,), jnp.float32),
        pltpu.SemaphoreType.DMA,
    ],
)(_bias_body)


@jax.jit
def kernel(u_idx, i_idx, mu, b_u, b_i, P, Q):
    u3 = u_idx.astype(jnp.int32).reshape(NW, NCHUNK, CHUNK)
    i3 = i_idx.astype(jnp.int32).reshape(NW, NCHUNK, CHUNK)
    pl_lin = _rp(P.T)
    tail = jnp.pad(P[NB_MAIN * W:, :].T, ((0, 0), (0, W - TAILW))).reshape(-1)
    pl_lin = lax.dynamic_update_slice(pl_lin, tail, (NB_MAIN * D * W,))
    dot = _mf(u3, i3, pl_lin, Q.T)
    bias = _bias(u3, i3, b_u, b_i)
    return dot + bias + mu


# speedup vs baseline: 15.4047x; 1.0028x over previous
import functools

import jax
import jax.numpy as jnp
from jax import lax
from jax.experimental import pallas as pl
from jax.experimental.pallas import tpu as pltpu
from jax.experimental.pallas import tpu_sc as plsc

B = 16384
D = 64
L = 16
NC = 2
NS = 16
NW = NC * NS
BPW = B // NW
CHUNK = 128
NCHUNK = BPW // CHUNK
LAG = 16

NU = 1000000
NI = 100000
W = 512
NB_MAIN = NU // W
TAILW = NU - NB_MAIN * W
NBLK = NB_MAIN + 1
BLK_PER_W = NB_MAIN // NW
LEN_P = NBLK * D * W


def _rp_body(pt_hbm, out_hbm, buf, semi, semo):
    wid = lax.axis_index("s") * NC + lax.axis_index("c")
    c0 = wid * BLK_PER_W

    def drain_outs():
        for _ in range(D):
            pltpu.make_async_copy(buf.at[0, 0],
                                  out_hbm.at[pl.ds(0, W)], semo).wait()

    def block(t, _):
        c = c0 + t
        tb = jnp.bitwise_and(t, 1)
        in_h = pltpu.async_copy(
            pt_hbm.at[:, pl.ds(pl.multiple_of(c * W, W), W)],
            buf.at[tb], semi)
        @pl.when(t > 0)
        def _():
            drain_outs()
        in_h.wait()
        base = c * (D * W)
        for kk in range(D):
            pltpu.async_copy(buf.at[tb, kk],
                             out_hbm.at[pl.ds(base + kk * W, W)], semo)
        return _

    lax.fori_loop(0, BLK_PER_W, block, None)
    drain_outs()

    @pl.when(wid == 0)
    def _():
        c = NB_MAIN - 1
        pltpu.sync_copy(pt_hbm.at[:, pl.ds(c * W, W)], buf.at[0])
        for kk in range(D):
            pltpu.async_copy(buf.at[0, kk],
                             out_hbm.at[pl.ds(c * D * W + kk * W, W)], semo)
        drain_outs()


_rp = functools.partial(
    pl.kernel,
    out_type=jax.ShapeDtypeStruct((LEN_P,), jnp.float32),
    mesh=plsc.VectorSubcoreMesh(core_axis_name="c", subcore_axis_name="s"),
    compiler_params=pltpu.CompilerParams(
        needs_layout_passes=False, use_tc_tiling_on_sc=True),
    scratch_types=[
        pltpu.VMEM((2, D, W), jnp.float32),
        pltpu.SemaphoreType.DMA,
        pltpu.SemaphoreType.DMA,
    ],
)(_rp_body)


def _mf_body(u_hbm, i_hbm, pl_hbm, qt_hbm, out_hbm,
             uidx_v, iidx_v, ubase_v, pv, qv, out_v, semp, semq):
    wid = lax.axis_index("s") * NC + lax.axis_index("c")
    base = wid * BPW

    pltpu.sync_copy(u_hbm.at[wid], uidx_v)
    pltpu.sync_copy(i_hbm.at[wid], iidx_v)

    sh9 = jnp.full((L,), 9, jnp.int32)
    sh15 = jnp.full((L,), 15, jnp.int32)
    m511 = jnp.full((L,), W - 1, jnp.int32)
    for c in range(NCHUNK):
        for g in range(CHUNK // L):
            sl = pl.ds(g * L, L)
            u = uidx_v[c, sl]
            ubase_v[c, sl] = lax.shift_left(
                lax.shift_right_logical(u, sh9), sh15
            ) + jnp.bitwise_and(u, m511)

    handles = [None] * D
    for k in range(D):
        ph = []
        for j in range(NCHUNK):
            sl = pl.ds(j * CHUNK, CHUNK)
            ph.append(pltpu.async_copy(
                pl_hbm.at[pl.ds(k * W, LEN_P - k * W)].at[ubase_v.at[j]],
                pv.at[k, sl], semp))
            ph.append(pltpu.async_copy(
                qt_hbm.at[k].at[iidx_v.at[j]], qv.at[k, sl], semq))
        handles[k] = ph
        if k >= LAG:
            for h in handles[k - LAG]:
                h.wait()
    for k in range(D - LAG, D):
        for h in handles[k]:
            h.wait()

    def group(g, _):
        sl = pl.ds(g * L, L)
        acc = pv[0, sl] * qv[0, sl]
        for k in range(1, D):
            acc = acc + pv[k, sl] * qv[k, sl]
        out_v[sl] = acc
        return _

    lax.fori_loop(0, BPW // L, group, None)

    pltpu.sync_copy(out_v, out_hbm.at[pl.ds(base, BPW)])


_mf = functools.partial(
    pl.kernel,
    out_type=jax.ShapeDtypeStruct((B,), jnp.float32),
    mesh=plsc.VectorSubcoreMesh(core_axis_name="c", subcore_axis_name="s"),
    compiler_params=pltpu.CompilerParams(
        needs_layout_passes=False, use_tc_tiling_on_sc=False),
    scratch_types=[
        pltpu.VMEM((NCHUNK, CHUNK), jnp.int32),
        pltpu.VMEM((NCHUNK, CHUNK), jnp.int32),
        pltpu.VMEM((NCHUNK, CHUNK), jnp.int32),
        pltpu.VMEM((D, BPW), jnp.float32),
        pltpu.VMEM((D, BPW), jnp.float32),
        pltpu.VMEM((BPW,), jnp.float32),
        pltpu.SemaphoreType.DMA,
        pltpu.SemaphoreType.DMA,
    ],
)(_mf_body)


def _bias_body(u_hbm, i_hbm, bu_hbm, bi_hbm, out_hbm,
               uidx_v, iidx_v, buv_v, biv_v, sem):
    wid = lax.axis_index("s") * NC + lax.axis_index("c")
    base = wid * BPW

    pltpu.sync_copy(u_hbm.at[wid], uidx_v)
    pltpu.sync_copy(i_hbm.at[wid], iidx_v)

    copies = []
    for j in range(NCHUNK):
        sl = pl.ds(j * CHUNK, CHUNK)
        copies.append(pltpu.async_copy(bu_hbm.at[uidx_v.at[j]], buv_v.at[sl], sem))
        copies.append(pltpu.async_copy(bi_hbm.at[iidx_v.at[j]], biv_v.at[sl], sem))
    for c in copies:
        c.wait()

    for g in range(BPW // L):
        sl = pl.ds(g * L, L)
        buv_v[sl] = buv_v[sl] + biv_v[sl]

    pltpu.sync_copy(buv_v, out_hbm.at[pl.ds(base, BPW)])


_bias = functools.partial(
    pl.kernel,
    out_type=jax.ShapeDtypeStruct((B,), jnp.float32),
    mesh=plsc.VectorSubcoreMesh(core_axis_name="c", subcore_axis_name="s"),
    compiler_params=pltpu.CompilerParams(
        needs_layout_passes=False, use_tc_tiling_on_sc=False),
    scratch_types=[
        pltpu.VMEM((NCHUNK, CHUNK), jnp.int32),
        pltpu.VMEM((NCHUNK, CHUNK), jnp.int32),
        pltpu.VMEM((BPW,), jnp.float32),
        pltpu.VMEM((BPW,), jnp.float32),
        pltpu.SemaphoreType.DMA,
    ],
)(_bias_body)


@jax.jit
def kernel(u_idx, i_idx, mu, b_u, b_i, P, Q):
    u3 = u_idx.astype(jnp.int32).reshape(NW, NCHUNK, CHUNK)
    i3 = i_idx.astype(jnp.int32).reshape(NW, NCHUNK, CHUNK)
    pl_lin = _rp(P.T)
    tail = jnp.pad(P[NB_MAIN * W:, :].T, ((0, 0), (0, W - TAILW))).reshape(-1)
    pl_lin = lax.dynamic_update_slice(pl_lin, tail, (NB_MAIN * D * W,))
    dot = _mf(u3, i3, pl_lin, Q.T)
    bias = _bias(u3, i3, b_u, b_i)
    return dot + bias + mu
